# baseline (speedup 1.0000x reference)
import functools

import jax
import jax.numpy as jnp
from jax import lax
from jax.experimental import pallas as pl
from jax.experimental.pallas import tpu as pltpu
from jax.experimental.pallas import tpu_sc as plsc

_B, _N, _D, _K = 16, 576, 384, 5
_L = 16
_NB = 8
_WPB = 4
_RPW = _N // _WPB
_GPW = _RPW // _L
_CH = 72
_NCH = _N // _CH
_IL = 3
_NP = 640
_BW = 256
_BSH = 128
_KS = 8
_WSTRIDE = _KS * _RPW
_BSTRIDE = _WPB * _WSTRIDE


def _sim_body(x_ref, s_ref):
    x = x_ref[0]
    norm = jnp.sqrt(jnp.sum(x * x, axis=1, keepdims=True)) + 1e-8
    xn = x / norm
    S = lax.dot_general(xn, xn, (((1,), (1,)), ((), ())),
                        preferred_element_type=jnp.float32)
    row = lax.broadcasted_iota(jnp.int32, (_N, _N), 0)
    col = lax.broadcasted_iota(jnp.int32, (_N, _N), 1)
    S = jnp.where(row == col, -1.5, S)
    s_ref[0] = jnp.concatenate(
        [S, jnp.zeros((_N, _NP - _N), jnp.float32)], axis=1)


def _chain5(k, ts):
    t1, t2, t3, t4, t5 = ts
    n1 = jnp.maximum(t1, k); k = jnp.minimum(t1, k)
    n2 = jnp.maximum(t2, k); k = jnp.minimum(t2, k)
    n3 = jnp.maximum(t3, k); k = jnp.minimum(t3, k)
    n4 = jnp.maximum(t4, k); k = jnp.minimum(t4, k)
    n5 = jnp.maximum(t5, k)
    return (n1, n2, n3, n4, n5)


def _sc_topk_body(s_hbm, idx_hbm, w_hbm, scol, kstate, idxbuf, wbuf,
                  sem0, sem1):
    wid = lax.axis_index("s") * 2 + lax.axis_index("c")
    b = lax.shift_right_logical(wid, 2)
    q = lax.bitwise_and(wid, 3)
    cb = q * _BSH
    nbase = q * _RPW - cb
    neg = jnp.full((_L,), jnp.int32(-0x7FFF0000), jnp.int32)
    zeroi = jnp.zeros((_L,), jnp.int32)
    ten = jnp.full((_L,), 10, jnp.int32)

    def init_body(g, carry):
        for k in range(_K):
            kstate[k, g] = neg
        return carry

    lax.fori_loop(0, _GPW, init_body, 0)

    sems = [sem0, sem1]
    handles = [None, None]
    handles[0] = pltpu.async_copy(
        s_hbm.at[b, pl.ds(0, _CH), pl.ds(cb, _BW)], scol.at[0], sems[0])
    for c in range(_NCH):
        if c + 1 < _NCH:
            nxt = (c + 1) & 1
            handles[nxt] = pltpu.async_copy(
                s_hbm.at[b, pl.ds((c + 1) * _CH, _CH), pl.ds(cb, _BW)],
                scol.at[nxt], sems[nxt])
        handles[c & 1].wait()
        buf = c & 1
        j0 = c * _CH

        def tri_body(t, carry2, buf=buf, j0=j0):
            g = t * _IL
            ts = []
            for q2 in range(_IL):
                ts += [kstate[k, g + q2] for k in range(_K)]

            def scan1(jj, s):
                i = zeroi + (1023 - j0 - jj)
                out = []
                for q2 in range(_IL):
                    n0 = nbase + (g + q2) * _L
                    v = scol[buf, jj, pl.ds(n0, _L)]
                    vi = lax.convert_element_type(v * 1048576.0, jnp.int32)
                    key = lax.shift_left(vi, ten) | i
                    out += list(_chain5(key, s[_K * q2:_K * q2 + _K]))
                return tuple(out)

            out = plsc.parallel_loop(0, _CH, carry=tuple(ts), unroll=2)(scan1)
            for q2 in range(_IL):
                for k in range(_K):
                    kstate[k, g + q2] = out[_K * q2 + k]
            return carry2

        lax.fori_loop(0, _GPW // _IL, tri_body, 0)

    def fin_body(g, carry):
        keys = [kstate[k, g] for k in range(_K)]
        vis = [lax.shift_right_arithmetic(keys[k], ten) for k in range(_K)]
        e = [jnp.exp(lax.convert_element_type(vis[k] - vis[0], jnp.float32)
                     * 9.5367431640625e-07) for k in range(_K)]
        invd = 1.0 / (e[0] + e[1] + e[2] + e[3] + e[4])
        ji = jnp.full((_L,), 1023, jnp.int32)
        for k in range(_K):
            idxbuf[pl.ds(k * _RPW + g * _L, _L)] = ji - (keys[k] & ji)
            wbuf[pl.ds(k * _RPW + g * _L, _L)] = e[k] * invd
        return carry

    lax.fori_loop(0, _GPW, fin_body, 0)
    off = wid * _WSTRIDE
    pltpu.sync_copy(idxbuf, idx_hbm.at[pl.ds(off, _WSTRIDE)])
    pltpu.sync_copy(wbuf, w_hbm.at[pl.ds(off, _WSTRIDE)])


_sc_topk = functools.partial(
    pl.kernel,
    out_type=(
        jax.ShapeDtypeStruct((_NB * _BSTRIDE,), jnp.int32),
        jax.ShapeDtypeStruct((_NB * _BSTRIDE,), jnp.float32),
    ),
    mesh=plsc.VectorSubcoreMesh(core_axis_name="c", subcore_axis_name="s"),
    scratch_types=[
        pltpu.VMEM((2, _CH, _BW), jnp.float32),
        pltpu.VMEM((_K, _GPW, _L), jnp.int32),
        pltpu.VMEM((_WSTRIDE,), jnp.int32),
        pltpu.VMEM((_WSTRIDE,), jnp.float32),
        pltpu.SemaphoreType.DMA,
        pltpu.SemaphoreType.DMA,
    ],
)(_sc_topk_body)


def _mlp_body(x_ref, idx_ref, w_ref, w1_ref, b1_ref, w2_ref, b2_ref, out_ref):
    x = x_ref[0]
    idxv = idx_ref[0, 0]
    wv = w_ref[0, 0]
    rowi = lax.broadcasted_iota(jnp.int32, (_N, _RPW), 0)
    aggs = []
    for q in range(_WPB):
        AT = jnp.zeros((_N, _RPW), jnp.float32)
        for k in range(_K):
            off = q * _WSTRIDE + k * _RPW
            ik = lax.slice(idxv, (off,), (off + _RPW,)).reshape(1, _RPW)
            wk = lax.slice(wv, (off,), (off + _RPW,)).reshape(1, _RPW)
            AT = AT + jnp.where(rowi == ik, wk, 0.0)
        aggs.append(lax.dot_general(AT, x, (((0,), (0,)), ((), ())),
                                    preferred_element_type=jnp.float32))
    agg = jnp.concatenate(aggs, axis=0)
    h = x + agg
    h1 = lax.dot_general(h, w1_ref[...], (((1,), (1,)), ((), ())),
                         preferred_element_type=jnp.float32)
    h1 = jnp.maximum(h1 + b1_ref[...], 0.0)
    h2 = lax.dot_general(h1, w2_ref[...], (((1,), (1,)), ((), ())),
                         preferred_element_type=jnp.float32)
    out_ref[0] = jnp.maximum(h2 + b2_ref[...], 0.0)


def _sim_stage(xc):
    return pl.pallas_call(
        _sim_body,
        grid=(_NB,),
        in_specs=[pl.BlockSpec((1, _N, _D), lambda b: (b, 0, 0))],
        out_specs=pl.BlockSpec((1, _N, _NP), lambda b: (b, 0, 0)),
        out_shape=jax.ShapeDtypeStruct((_NB, _N, _NP), jnp.float32),
        compiler_params=pltpu.CompilerParams(
            dimension_semantics=("arbitrary",),
        ),
    )(xc)


def _mlp_stage(xc, idx_flat, w_flat, W1, b1r, W2, b2r):
    idx2 = idx_flat.reshape(_NB, 1, _BSTRIDE)
    w2 = w_flat.reshape(_NB, 1, _BSTRIDE)
    H = W1.shape[0]
    return pl.pallas_call(
        _mlp_body,
        grid=(_NB,),
        in_specs=[
            pl.BlockSpec((1, _N, _D), lambda b: (b, 0, 0)),
            pl.BlockSpec((1, 1, _BSTRIDE), lambda b: (b, 0, 0)),
            pl.BlockSpec((1, 1, _BSTRIDE), lambda b: (b, 0, 0)),
            pl.BlockSpec((H, _D), lambda b: (0, 0)),
            pl.BlockSpec((1, H), lambda b: (0, 0)),
            pl.BlockSpec((H, H), lambda b: (0, 0)),
            pl.BlockSpec((1, H), lambda b: (0, 0)),
        ],
        out_specs=pl.BlockSpec((1, _N, H), lambda b: (b, 0, 0)),
        out_shape=jax.ShapeDtypeStruct((_NB, _N, H), jnp.float32),
        compiler_params=pltpu.CompilerParams(
            dimension_semantics=("arbitrary",),
        ),
    )(xc, idx2, w2, W1, b1r, W2, b2r)


@jax.jit
def kernel(x, W1, b1, W2, b2):
    H = W1.shape[0]
    b1r = b1.reshape(1, H)
    b2r = b2.reshape(1, H)
    xa, xb = x[:_NB], x[_NB:]
    Sa = _sim_stage(xa)
    ia, wa = _sc_topk(Sa)
    Sb = _sim_stage(xb)
    ib, wb = _sc_topk(Sb)
    outa = _mlp_stage(xa, ia, wa, W1, b1r, W2, b2r)
    outb = _mlp_stage(xb, ib, wb, W1, b1r, W2, b2r)
    return jnp.concatenate([outa, outb], axis=0)


# speedup vs baseline: 1.0128x; 1.0128x over previous
import functools

import jax
import jax.numpy as jnp
from jax import lax
from jax.experimental import pallas as pl
from jax.experimental.pallas import tpu as pltpu
from jax.experimental.pallas import tpu_sc as plsc

_B, _N, _D, _K = 16, 576, 384, 5
_L = 16
_NB = 8
_WPB = 4
_RPW = _N // _WPB
_GPW = _RPW // _L
_CH = 72
_NCH = _N // _CH
_IL = 3
_NP = 640
_BW = 256
_BSH = 128
_KS = 8
_WSTRIDE = _KS * _RPW
_BSTRIDE = _WPB * _WSTRIDE


def _sim_body(x_ref, s_ref):
    x = x_ref[0]
    norm = jnp.sqrt(jnp.sum(x * x, axis=1, keepdims=True)) + 1e-8
    xn = x / norm
    S = lax.dot_general(xn, xn, (((1,), (1,)), ((), ())),
                        preferred_element_type=jnp.float32)
    row = lax.broadcasted_iota(jnp.int32, (_N, _N), 0)
    col = lax.broadcasted_iota(jnp.int32, (_N, _N), 1)
    S = jnp.where(row == col, -1.5, S)
    s_ref[0] = jnp.concatenate(
        [S, jnp.zeros((_N, _NP - _N), jnp.float32)], axis=1)


def _chain5(k, ts):
    t1, t2, t3, t4, t5 = ts
    n1 = jnp.maximum(t1, k); k = jnp.minimum(t1, k)
    n2 = jnp.maximum(t2, k); k = jnp.minimum(t2, k)
    n3 = jnp.maximum(t3, k); k = jnp.minimum(t3, k)
    n4 = jnp.maximum(t4, k); k = jnp.minimum(t4, k)
    n5 = jnp.maximum(t5, k)
    return (n1, n2, n3, n4, n5)


def _sc_topk_body(off, s_hbm, idx_hbm, w_hbm, scol, kstate, idxbuf, wbuf,
                  sem0, sem1):
    wid = lax.axis_index("s") * 2 + lax.axis_index("c")
    b = off + lax.shift_right_logical(wid, 2)
    q = lax.bitwise_and(wid, 3)
    cb = q * _BSH
    nbase = q * _RPW - cb
    neg = jnp.full((_L,), jnp.int32(-0x7FFF0000), jnp.int32)
    zeroi = jnp.zeros((_L,), jnp.int32)
    ten = jnp.full((_L,), 10, jnp.int32)

    def init_body(g, carry):
        for k in range(_K):
            kstate[k, g] = neg
        return carry

    lax.fori_loop(0, _GPW, init_body, 0)

    sems = [sem0, sem1]
    handles = [None, None]
    handles[0] = pltpu.async_copy(
        s_hbm.at[b, pl.ds(0, _CH), pl.ds(cb, _BW)], scol.at[0], sems[0])
    for c in range(_NCH):
        if c + 1 < _NCH:
            nxt = (c + 1) & 1
            handles[nxt] = pltpu.async_copy(
                s_hbm.at[b, pl.ds((c + 1) * _CH, _CH), pl.ds(cb, _BW)],
                scol.at[nxt], sems[nxt])
        handles[c & 1].wait()
        buf = c & 1
        j0 = c * _CH

        def tri_body(t, carry2, buf=buf, j0=j0):
            g = t * _IL
            ts = []
            for q2 in range(_IL):
                ts += [kstate[k, g + q2] for k in range(_K)]

            def scan1(jj, s):
                i = zeroi + (1023 - j0 - jj)
                out = []
                for q2 in range(_IL):
                    n0 = nbase + (g + q2) * _L
                    v = scol[buf, jj, pl.ds(n0, _L)]
                    vi = lax.convert_element_type(v * 1048576.0, jnp.int32)
                    key = lax.shift_left(vi, ten) | i
                    out += list(_chain5(key, s[_K * q2:_K * q2 + _K]))
                return tuple(out)

            out = plsc.parallel_loop(0, _CH, carry=tuple(ts), unroll=2)(scan1)
            for q2 in range(_IL):
                for k in range(_K):
                    kstate[k, g + q2] = out[_K * q2 + k]
            return carry2

        lax.fori_loop(0, _GPW // _IL, tri_body, 0)

    def fin_body(g, carry):
        keys = [kstate[k, g] for k in range(_K)]
        vis = [lax.shift_right_arithmetic(keys[k], ten) for k in range(_K)]
        e = [jnp.exp(lax.convert_element_type(vis[k] - vis[0], jnp.float32)
                     * 9.5367431640625e-07) for k in range(_K)]
        invd = 1.0 / (e[0] + e[1] + e[2] + e[3] + e[4])
        ji = jnp.full((_L,), 1023, jnp.int32)
        for k in range(_K):
            idxbuf[pl.ds(k * _RPW + g * _L, _L)] = ji - (keys[k] & ji)
            wbuf[pl.ds(k * _RPW + g * _L, _L)] = e[k] * invd
        return carry

    lax.fori_loop(0, _GPW, fin_body, 0)
    off = wid * _WSTRIDE
    pltpu.sync_copy(idxbuf, idx_hbm.at[pl.ds(off, _WSTRIDE)])
    pltpu.sync_copy(wbuf, w_hbm.at[pl.ds(off, _WSTRIDE)])


def _make_sc_topk(off):
    return functools.partial(
        pl.kernel,
        out_type=(
            jax.ShapeDtypeStruct((_NB * _BSTRIDE,), jnp.int32),
            jax.ShapeDtypeStruct((_NB * _BSTRIDE,), jnp.float32),
        ),
        mesh=plsc.VectorSubcoreMesh(core_axis_name="c", subcore_axis_name="s"),
        scratch_types=[
            pltpu.VMEM((2, _CH, _BW), jnp.float32),
            pltpu.VMEM((_K, _GPW, _L), jnp.int32),
            pltpu.VMEM((_WSTRIDE,), jnp.int32),
            pltpu.VMEM((_WSTRIDE,), jnp.float32),
            pltpu.SemaphoreType.DMA,
            pltpu.SemaphoreType.DMA,
        ],
    )(functools.partial(_sc_topk_body, off))


_sc_topk_a = _make_sc_topk(0)
_sc_topk_b = _make_sc_topk(_NB)


def _mlp_body(x_ref, idx_ref, w_ref, w1_ref, b1_ref, w2_ref, b2_ref, out_ref):
    x = x_ref[0]
    idxv = idx_ref[0, 0]
    wv = w_ref[0, 0]
    rowi = lax.broadcasted_iota(jnp.int32, (_N, _RPW), 0)
    aggs = []
    for q in range(_WPB):
        AT = jnp.zeros((_N, _RPW), jnp.float32)
        for k in range(_K):
            off = q * _WSTRIDE + k * _RPW
            ik = lax.slice(idxv, (off,), (off + _RPW,)).reshape(1, _RPW)
            wk = lax.slice(wv, (off,), (off + _RPW,)).reshape(1, _RPW)
            AT = AT + jnp.where(rowi == ik, wk, 0.0)
        aggs.append(lax.dot_general(AT, x, (((0,), (0,)), ((), ())),
                                    preferred_element_type=jnp.float32))
    agg = jnp.concatenate(aggs, axis=0)
    h = x + agg
    h1 = lax.dot_general(h, w1_ref[...], (((1,), (1,)), ((), ())),
                         preferred_element_type=jnp.float32)
    h1 = jnp.maximum(h1 + b1_ref[...], 0.0)
    h2 = lax.dot_general(h1, w2_ref[...], (((1,), (1,)), ((), ())),
                         preferred_element_type=jnp.float32)
    out_ref[0] = jnp.maximum(h2 + b2_ref[...], 0.0)


def _sim_stage(xc):
    return pl.pallas_call(
        _sim_body,
        grid=(_B,),
        in_specs=[pl.BlockSpec((1, _N, _D), lambda b: (b, 0, 0))],
        out_specs=pl.BlockSpec((1, _N, _NP), lambda b: (b, 0, 0)),
        out_shape=jax.ShapeDtypeStruct((_B, _N, _NP), jnp.float32),
        compiler_params=pltpu.CompilerParams(
            dimension_semantics=("arbitrary",),
        ),
    )(xc)


def _mlp_stage(xc, idx_flat, w_flat, W1, b1r, W2, b2r):
    idx2 = idx_flat.reshape(_NB, 1, _BSTRIDE)
    w2 = w_flat.reshape(_NB, 1, _BSTRIDE)
    H = W1.shape[0]
    return pl.pallas_call(
        _mlp_body,
        grid=(_NB,),
        in_specs=[
            pl.BlockSpec((1, _N, _D), lambda b: (b, 0, 0)),
            pl.BlockSpec((1, 1, _BSTRIDE), lambda b: (b, 0, 0)),
            pl.BlockSpec((1, 1, _BSTRIDE), lambda b: (b, 0, 0)),
            pl.BlockSpec((H, _D), lambda b: (0, 0)),
            pl.BlockSpec((1, H), lambda b: (0, 0)),
            pl.BlockSpec((H, H), lambda b: (0, 0)),
            pl.BlockSpec((1, H), lambda b: (0, 0)),
        ],
        out_specs=pl.BlockSpec((1, _N, H), lambda b: (b, 0, 0)),
        out_shape=jax.ShapeDtypeStruct((_NB, _N, H), jnp.float32),
        compiler_params=pltpu.CompilerParams(
            dimension_semantics=("arbitrary",),
        ),
    )(xc, idx2, w2, W1, b1r, W2, b2r)


@jax.jit
def kernel(x, W1, b1, W2, b2):
    H = W1.shape[0]
    b1r = b1.reshape(1, H)
    b2r = b2.reshape(1, H)
    xa, xb = x[:_NB], x[_NB:]
    S = _sim_stage(x)
    ia, wa = _sc_topk_a(S)
    ib, wb = _sc_topk_b(S)
    outa = _mlp_stage(xa, ia, wa, W1, b1r, W2, b2r)
    outb = _mlp_stage(xb, ib, wb, W1, b1r, W2, b2r)
    return jnp.concatenate([outa, outb], axis=0)


# speedup vs baseline: 1.1057x; 1.0917x over previous
import functools

import jax
import jax.numpy as jnp
from jax import lax
from jax.experimental import pallas as pl
from jax.experimental.pallas import tpu as pltpu
from jax.experimental.pallas import tpu_sc as plsc

_B, _N, _D, _K = 16, 576, 384, 5
_L = 16
_NB = 8
_WPB = 4
_RPW = _N // _WPB
_GPW = _RPW // _L
_CH = 72
_NCH = _N // _CH
_IL = 3
_NP = 640
_BW = 256
_BSH = 128
_KS = 8
_WSTRIDE = _KS * _RPW
_BSTRIDE = _WPB * _WSTRIDE


def _sim_body(x_ref, s_ref):
    x = x_ref[0]
    norm = jnp.sqrt(jnp.sum(x * x, axis=1, keepdims=True)) + 1e-8
    xn = x / norm
    S = lax.dot_general(xn, xn, (((1,), (1,)), ((), ())),
                        preferred_element_type=jnp.float32)
    row = lax.broadcasted_iota(jnp.int32, (_N, _N), 0)
    col = lax.broadcasted_iota(jnp.int32, (_N, _N), 1)
    S = jnp.where(row == col, -1.5, S)
    s_ref[0] = jnp.concatenate(
        [S, jnp.zeros((_N, _NP - _N), jnp.float32)], axis=1)


def _chain5(k, ts):
    t1, t2, t3, t4, t5 = ts
    n1 = jnp.maximum(t1, k); k = jnp.minimum(t1, k)
    n2 = jnp.maximum(t2, k); k = jnp.minimum(t2, k)
    n3 = jnp.maximum(t3, k); k = jnp.minimum(t3, k)
    n4 = jnp.maximum(t4, k); k = jnp.minimum(t4, k)
    n5 = jnp.maximum(t5, k)
    return (n1, n2, n3, n4, n5)


def _sc_topk_body(off, s_hbm, idx_hbm, w_hbm, scol, kstate, idxbuf, wbuf,
                  sem0, sem1):
    wid = lax.axis_index("s") * 2 + lax.axis_index("c")
    b = off + lax.shift_right_logical(wid, 2)
    q = lax.bitwise_and(wid, 3)
    cb = q * _BSH
    nbase = q * _RPW - cb
    neg = jnp.full((_L,), jnp.int32(-0x7FFF0000), jnp.int32)
    zeroi = jnp.zeros((_L,), jnp.int32)
    ten = jnp.full((_L,), 10, jnp.int32)

    def init_body(g, carry):
        for k in range(_K):
            kstate[k, g] = neg
        return carry

    lax.fori_loop(0, _GPW, init_body, 0)

    sems = [sem0, sem1]
    handles = [None, None]
    handles[0] = pltpu.async_copy(
        s_hbm.at[b, pl.ds(0, _CH), pl.ds(cb, _BW)], scol.at[0], sems[0])
    for c in range(_NCH):
        if c + 1 < _NCH:
            nxt = (c + 1) & 1
            handles[nxt] = pltpu.async_copy(
                s_hbm.at[b, pl.ds((c + 1) * _CH, _CH), pl.ds(cb, _BW)],
                scol.at[nxt], sems[nxt])
        handles[c & 1].wait()
        buf = c & 1
        j0 = c * _CH

        def tri_body(t, carry2, buf=buf, j0=j0):
            g = t * _IL
            ts = []
            for q2 in range(_IL):
                ts += [kstate[k, g + q2] for k in range(_K)]

            def scan1(jj, s):
                i = zeroi + (1023 - j0 - jj)
                out = []
                for q2 in range(_IL):
                    n0 = nbase + (g + q2) * _L
                    v = scol[buf, jj, pl.ds(n0, _L)]
                    vi = lax.convert_element_type(v * 1048576.0, jnp.int32)
                    key = lax.shift_left(vi, ten) | i
                    out += list(_chain5(key, s[_K * q2:_K * q2 + _K]))
                return tuple(out)

            out = plsc.parallel_loop(0, _CH, carry=tuple(ts), unroll=2)(scan1)
            for q2 in range(_IL):
                for k in range(_K):
                    kstate[k, g + q2] = out[_K * q2 + k]
            return carry2

        lax.fori_loop(0, _GPW // _IL, tri_body, 0)

    def fin_body(g, carry):
        keys = [kstate[k, g] for k in range(_K)]
        vis = [lax.shift_right_arithmetic(keys[k], ten) for k in range(_K)]
        e = [jnp.exp(lax.convert_element_type(vis[k] - vis[0], jnp.float32)
                     * 9.5367431640625e-07) for k in range(_K)]
        invd = 1.0 / (e[0] + e[1] + e[2] + e[3] + e[4])
        ji = jnp.full((_L,), 1023, jnp.int32)
        for k in range(_K):
            idxbuf[pl.ds(k * _RPW + g * _L, _L)] = ji - (keys[k] & ji)
            wbuf[pl.ds(k * _RPW + g * _L, _L)] = e[k] * invd
        return carry

    lax.fori_loop(0, _GPW, fin_body, 0)
    off = wid * _WSTRIDE
    pltpu.sync_copy(idxbuf, idx_hbm.at[pl.ds(off, _WSTRIDE)])
    pltpu.sync_copy(wbuf, w_hbm.at[pl.ds(off, _WSTRIDE)])


def _make_sc_topk(off):
    return functools.partial(
        pl.kernel,
        out_type=(
            jax.ShapeDtypeStruct((_NB * _BSTRIDE,), jnp.int32),
            jax.ShapeDtypeStruct((_NB * _BSTRIDE,), jnp.float32),
        ),
        mesh=plsc.VectorSubcoreMesh(core_axis_name="c", subcore_axis_name="s"),
        scratch_types=[
            pltpu.VMEM((2, _CH, _BW), jnp.float32),
            pltpu.VMEM((_K, _GPW, _L), jnp.int32),
            pltpu.VMEM((_WSTRIDE,), jnp.int32),
            pltpu.VMEM((_WSTRIDE,), jnp.float32),
            pltpu.SemaphoreType.DMA,
            pltpu.SemaphoreType.DMA,
        ],
    )(functools.partial(_sc_topk_body, off))


_sc_topk_a = _make_sc_topk(0)
_sc_topk_b = _make_sc_topk(_NB)


def _mlp_body(x_ref, idx_ref, w_ref, w1_ref, b1_ref, w2_ref, b2_ref, out_ref):
    x = x_ref[0]
    idxv = idx_ref[0, 0]
    wv = w_ref[0, 0]
    rowi = lax.broadcasted_iota(jnp.int32, (_N, _RPW), 0)
    aggs = []
    for q in range(_WPB):
        AT = jnp.zeros((_N, _RPW), jnp.float32)
        for k in range(_K):
            off = q * _WSTRIDE + k * _RPW
            ik = lax.slice(idxv, (off,), (off + _RPW,)).reshape(1, _RPW)
            wk = lax.slice(wv, (off,), (off + _RPW,)).reshape(1, _RPW)
            AT = AT + jnp.where(rowi == ik, wk, 0.0)
        aggs.append(lax.dot_general(AT, x, (((0,), (0,)), ((), ())),
                                    preferred_element_type=jnp.float32))
    agg = jnp.concatenate(aggs, axis=0)
    h = x + agg
    h1 = lax.dot_general(h, w1_ref[...], (((1,), (1,)), ((), ())),
                         preferred_element_type=jnp.float32)
    h1 = jnp.maximum(h1 + b1_ref[...], 0.0)
    h2 = lax.dot_general(h1, w2_ref[...], (((1,), (1,)), ((), ())),
                         preferred_element_type=jnp.float32)
    out_ref[0] = jnp.maximum(h2 + b2_ref[...], 0.0)


def _sim_stage(xc):
    return pl.pallas_call(
        _sim_body,
        grid=(_B,),
        in_specs=[pl.BlockSpec((1, _N, _D), lambda b: (b, 0, 0))],
        out_specs=pl.BlockSpec((1, _N, _NP), lambda b: (b, 0, 0)),
        out_shape=jax.ShapeDtypeStruct((_B, _N, _NP), jnp.float32),
        compiler_params=pltpu.CompilerParams(
            dimension_semantics=("arbitrary",),
        ),
    )(xc)


def _mlp_body_b(prev_ref, x_ref, idx_ref, w_ref, w1_ref, b1_ref, w2_ref,
                b2_ref, out_ref):
    del prev_ref
    _mlp_body(x_ref, idx_ref, w_ref, w1_ref, b1_ref, w2_ref, b2_ref, out_ref)


def _mlp_stage_a(xc, idx_flat, w_flat, W1, b1r, W2, b2r):
    idx2 = idx_flat.reshape(_NB, 1, _BSTRIDE)
    w2 = w_flat.reshape(_NB, 1, _BSTRIDE)
    H = W1.shape[0]
    return pl.pallas_call(
        _mlp_body,
        grid=(_NB,),
        in_specs=[
            pl.BlockSpec((1, _N, _D), lambda b: (b, 0, 0)),
            pl.BlockSpec((1, 1, _BSTRIDE), lambda b: (b, 0, 0)),
            pl.BlockSpec((1, 1, _BSTRIDE), lambda b: (b, 0, 0)),
            pl.BlockSpec((H, _D), lambda b: (0, 0)),
            pl.BlockSpec((1, H), lambda b: (0, 0)),
            pl.BlockSpec((H, H), lambda b: (0, 0)),
            pl.BlockSpec((1, H), lambda b: (0, 0)),
        ],
        out_specs=pl.BlockSpec((1, _N, H), lambda b: (b, 0, 0)),
        out_shape=jax.ShapeDtypeStruct((_B, _N, H), jnp.float32),
        compiler_params=pltpu.CompilerParams(
            dimension_semantics=("arbitrary",),
        ),
    )(xc, idx2, w2, W1, b1r, W2, b2r)


def _mlp_stage_b(prev, xc, idx_flat, w_flat, W1, b1r, W2, b2r):
    idx2 = idx_flat.reshape(_NB, 1, _BSTRIDE)
    w2 = w_flat.reshape(_NB, 1, _BSTRIDE)
    H = W1.shape[0]
    return pl.pallas_call(
        _mlp_body_b,
        grid=(_NB,),
        in_specs=[
            pl.BlockSpec(memory_space=pl.ANY),
            pl.BlockSpec((1, _N, _D), lambda b: (b, 0, 0)),
            pl.BlockSpec((1, 1, _BSTRIDE), lambda b: (b, 0, 0)),
            pl.BlockSpec((1, 1, _BSTRIDE), lambda b: (b, 0, 0)),
            pl.BlockSpec((H, _D), lambda b: (0, 0)),
            pl.BlockSpec((1, H), lambda b: (0, 0)),
            pl.BlockSpec((H, H), lambda b: (0, 0)),
            pl.BlockSpec((1, H), lambda b: (0, 0)),
        ],
        out_specs=pl.BlockSpec((1, _N, H), lambda b: (b + _NB, 0, 0)),
        out_shape=jax.ShapeDtypeStruct((_B, _N, H), jnp.float32),
        input_output_aliases={0: 0},
        compiler_params=pltpu.CompilerParams(
            dimension_semantics=("arbitrary",),
        ),
    )(prev, xc, idx2, w2, W1, b1r, W2, b2r)


@jax.jit
def kernel(x, W1, b1, W2, b2):
    H = W1.shape[0]
    b1r = b1.reshape(1, H)
    b2r = b2.reshape(1, H)
    xa, xb = x[:_NB], x[_NB:]
    S = _sim_stage(x)
    ia, wa = _sc_topk_a(S)
    ib, wb = _sc_topk_b(S)
    outa = _mlp_stage_a(xa, ia, wa, W1, b1r, W2, b2r)
    return _mlp_stage_b(outa, xb, ib, wb, W1, b1r, W2, b2r)


# speedup vs baseline: 1.2047x; 1.0895x over previous
import functools

import jax
import jax.numpy as jnp
from jax import lax
from jax.experimental import pallas as pl
from jax.experimental.pallas import tpu as pltpu
from jax.experimental.pallas import tpu_sc as plsc

_B, _N, _D, _K = 16, 576, 384, 5
_L = 16
_NB = 8
_WPB = 4
_RPW = _N // _WPB
_GPW = _RPW // _L
_CH = 72
_NCH = _N // _CH
_IL = 3
_NP = 640
_BW = 256
_BSH = 128
_KS = 8
_WSTRIDE = _KS * _RPW
_BSTRIDE = _WPB * _WSTRIDE


def _sim_body(x_ref, s_ref):
    x = x_ref[0]
    norm = jnp.sqrt(jnp.sum(x * x, axis=1, keepdims=True)) + 1e-8
    xn = x / norm
    S = lax.dot_general(xn, xn, (((1,), (1,)), ((), ())),
                        preferred_element_type=jnp.float32)
    row = lax.broadcasted_iota(jnp.int32, (_N, _N), 0)
    col = lax.broadcasted_iota(jnp.int32, (_N, _N), 1)
    vi = lax.convert_element_type(S * 1048576.0, jnp.int32)
    key = vi * 1024 + (1023 - col)
    key = jnp.where(row == col, -0x7FFF0000, key)
    s_ref[0] = jnp.concatenate(
        [key, jnp.zeros((_N, _NP - _N), jnp.int32)], axis=1)


def _chain5(k, ts):
    t1, t2, t3, t4, t5 = ts
    n1 = jnp.maximum(t1, k); k = jnp.minimum(t1, k)
    n2 = jnp.maximum(t2, k); k = jnp.minimum(t2, k)
    n3 = jnp.maximum(t3, k); k = jnp.minimum(t3, k)
    n4 = jnp.maximum(t4, k); k = jnp.minimum(t4, k)
    n5 = jnp.maximum(t5, k)
    return (n1, n2, n3, n4, n5)


def _sc_topk_body(off, s_hbm, idx_hbm, w_hbm, scol, kstate, idxbuf, wbuf,
                  sem0, sem1):
    wid = lax.axis_index("s") * 2 + lax.axis_index("c")
    b = off + lax.shift_right_logical(wid, 2)
    q = lax.bitwise_and(wid, 3)
    cb = q * _BSH
    nbase = q * _RPW - cb
    neg = jnp.full((_L,), jnp.int32(-0x7FFF0000), jnp.int32)
    zeroi = jnp.zeros((_L,), jnp.int32)
    ten = jnp.full((_L,), 10, jnp.int32)

    def init_body(g, carry):
        for k in range(_K):
            kstate[k, g] = neg
        return carry

    lax.fori_loop(0, _GPW, init_body, 0)

    sems = [sem0, sem1]
    handles = [None, None]
    handles[0] = pltpu.async_copy(
        s_hbm.at[b, pl.ds(0, _CH), pl.ds(cb, _BW)], scol.at[0], sems[0])
    for c in range(_NCH):
        if c + 1 < _NCH:
            nxt = (c + 1) & 1
            handles[nxt] = pltpu.async_copy(
                s_hbm.at[b, pl.ds((c + 1) * _CH, _CH), pl.ds(cb, _BW)],
                scol.at[nxt], sems[nxt])
        handles[c & 1].wait()
        buf = c & 1
        j0 = c * _CH

        def tri_body(t, carry2, buf=buf, j0=j0):
            g = t * _IL
            ts = []
            for q2 in range(_IL):
                ts += [kstate[k, g + q2] for k in range(_K)]

            def scan1(jj, s):
                out = []
                for q2 in range(_IL):
                    n0 = nbase + (g + q2) * _L
                    key = scol[buf, jj, pl.ds(n0, _L)]
                    out += list(_chain5(key, s[_K * q2:_K * q2 + _K]))
                return tuple(out)

            out = plsc.parallel_loop(0, _CH, carry=tuple(ts), unroll=2)(scan1)
            for q2 in range(_IL):
                for k in range(_K):
                    kstate[k, g + q2] = out[_K * q2 + k]
            return carry2

        lax.fori_loop(0, _GPW // _IL, tri_body, 0)

    def fin_body(g, carry):
        keys = [kstate[k, g] for k in range(_K)]
        vis = [lax.shift_right_arithmetic(keys[k], ten) for k in range(_K)]
        e = [jnp.exp(lax.convert_element_type(vis[k] - vis[0], jnp.float32)
                     * 9.5367431640625e-07) for k in range(_K)]
        invd = 1.0 / (e[0] + e[1] + e[2] + e[3] + e[4])
        ji = jnp.full((_L,), 1023, jnp.int32)
        for k in range(_K):
            idxbuf[pl.ds(k * _RPW + g * _L, _L)] = ji - (keys[k] & ji)
            wbuf[pl.ds(k * _RPW + g * _L, _L)] = e[k] * invd
        return carry

    lax.fori_loop(0, _GPW, fin_body, 0)
    off = wid * _WSTRIDE
    pltpu.sync_copy(idxbuf, idx_hbm.at[pl.ds(off, _WSTRIDE)])
    pltpu.sync_copy(wbuf, w_hbm.at[pl.ds(off, _WSTRIDE)])


def _make_sc_topk(off):
    return functools.partial(
        pl.kernel,
        out_type=(
            jax.ShapeDtypeStruct((_NB * _BSTRIDE,), jnp.int32),
            jax.ShapeDtypeStruct((_NB * _BSTRIDE,), jnp.float32),
        ),
        mesh=plsc.VectorSubcoreMesh(core_axis_name="c", subcore_axis_name="s"),
        scratch_types=[
            pltpu.VMEM((2, _CH, _BW), jnp.int32),
            pltpu.VMEM((_K, _GPW, _L), jnp.int32),
            pltpu.VMEM((_WSTRIDE,), jnp.int32),
            pltpu.VMEM((_WSTRIDE,), jnp.float32),
            pltpu.SemaphoreType.DMA,
            pltpu.SemaphoreType.DMA,
        ],
    )(functools.partial(_sc_topk_body, off))


_sc_topk_a = _make_sc_topk(0)
_sc_topk_b = _make_sc_topk(_NB)


def _mlp_body(x_ref, idx_ref, w_ref, w1_ref, b1_ref, w2_ref, b2_ref, out_ref):
    x = x_ref[0]
    idxv = idx_ref[0, 0]
    wv = w_ref[0, 0]
    rowi = lax.broadcasted_iota(jnp.int32, (_N, _RPW), 0)
    aggs = []
    for q in range(_WPB):
        AT = jnp.zeros((_N, _RPW), jnp.float32)
        for k in range(_K):
            off = q * _WSTRIDE + k * _RPW
            ik = lax.slice(idxv, (off,), (off + _RPW,)).reshape(1, _RPW)
            wk = lax.slice(wv, (off,), (off + _RPW,)).reshape(1, _RPW)
            AT = AT + jnp.where(rowi == ik, wk, 0.0)
        aggs.append(lax.dot_general(AT, x, (((0,), (0,)), ((), ())),
                                    preferred_element_type=jnp.float32))
    agg = jnp.concatenate(aggs, axis=0)
    h = x + agg
    h1 = lax.dot_general(h, w1_ref[...], (((1,), (1,)), ((), ())),
                         preferred_element_type=jnp.float32)
    h1 = jnp.maximum(h1 + b1_ref[...], 0.0)
    h2 = lax.dot_general(h1, w2_ref[...], (((1,), (1,)), ((), ())),
                         preferred_element_type=jnp.float32)
    out_ref[0] = jnp.maximum(h2 + b2_ref[...], 0.0)


def _sim_stage(xc):
    return pl.pallas_call(
        _sim_body,
        grid=(_B,),
        in_specs=[pl.BlockSpec((1, _N, _D), lambda b: (b, 0, 0))],
        out_specs=pl.BlockSpec((1, _N, _NP), lambda b: (b, 0, 0)),
        out_shape=jax.ShapeDtypeStruct((_B, _N, _NP), jnp.int32),
        compiler_params=pltpu.CompilerParams(
            dimension_semantics=("arbitrary",),
        ),
    )(xc)


def _mlp_body_b(prev_ref, x_ref, idx_ref, w_ref, w1_ref, b1_ref, w2_ref,
                b2_ref, out_ref):
    del prev_ref
    _mlp_body(x_ref, idx_ref, w_ref, w1_ref, b1_ref, w2_ref, b2_ref, out_ref)


def _mlp_stage_a(xc, idx_flat, w_flat, W1, b1r, W2, b2r):
    idx2 = idx_flat.reshape(_NB, 1, _BSTRIDE)
    w2 = w_flat.reshape(_NB, 1, _BSTRIDE)
    H = W1.shape[0]
    return pl.pallas_call(
        _mlp_body,
        grid=(_NB,),
        in_specs=[
            pl.BlockSpec((1, _N, _D), lambda b: (b, 0, 0)),
            pl.BlockSpec((1, 1, _BSTRIDE), lambda b: (b, 0, 0)),
            pl.BlockSpec((1, 1, _BSTRIDE), lambda b: (b, 0, 0)),
            pl.BlockSpec((H, _D), lambda b: (0, 0)),
            pl.BlockSpec((1, H), lambda b: (0, 0)),
            pl.BlockSpec((H, H), lambda b: (0, 0)),
            pl.BlockSpec((1, H), lambda b: (0, 0)),
        ],
        out_specs=pl.BlockSpec((1, _N, H), lambda b: (b, 0, 0)),
        out_shape=jax.ShapeDtypeStruct((_B, _N, H), jnp.float32),
        compiler_params=pltpu.CompilerParams(
            dimension_semantics=("arbitrary",),
        ),
    )(xc, idx2, w2, W1, b1r, W2, b2r)


def _mlp_stage_b(prev, xc, idx_flat, w_flat, W1, b1r, W2, b2r):
    idx2 = idx_flat.reshape(_NB, 1, _BSTRIDE)
    w2 = w_flat.reshape(_NB, 1, _BSTRIDE)
    H = W1.shape[0]
    return pl.pallas_call(
        _mlp_body_b,
        grid=(_NB,),
        in_specs=[
            pl.BlockSpec(memory_space=pl.ANY),
            pl.BlockSpec((1, _N, _D), lambda b: (b, 0, 0)),
            pl.BlockSpec((1, 1, _BSTRIDE), lambda b: (b, 0, 0)),
            pl.BlockSpec((1, 1, _BSTRIDE), lambda b: (b, 0, 0)),
            pl.BlockSpec((H, _D), lambda b: (0, 0)),
            pl.BlockSpec((1, H), lambda b: (0, 0)),
            pl.BlockSpec((H, H), lambda b: (0, 0)),
            pl.BlockSpec((1, H), lambda b: (0, 0)),
        ],
        out_specs=pl.BlockSpec((1, _N, H), lambda b: (b + _NB, 0, 0)),
        out_shape=jax.ShapeDtypeStruct((_B, _N, H), jnp.float32),
        input_output_aliases={0: 0},
        compiler_params=pltpu.CompilerParams(
            dimension_semantics=("arbitrary",),
        ),
    )(prev, xc, idx2, w2, W1, b1r, W2, b2r)


@jax.jit
def kernel(x, W1, b1, W2, b2):
    H = W1.shape[0]
    b1r = b1.reshape(1, H)
    b2r = b2.reshape(1, H)
    xa, xb = x[:_NB], x[_NB:]
    S = _sim_stage(x)
    ia, wa = _sc_topk_a(S)
    ib, wb = _sc_topk_b(S)
    outa = _mlp_stage_a(xa, ia, wa, W1, b1r, W2, b2r)
    return _mlp_stage_b(outa, xb, ib, wb, W1, b1r, W2, b2r)
